# baseline (speedup 1.0000x reference)
import functools

import jax
import jax.numpy as jnp
import numpy as np
from jax import lax
from jax.experimental import pallas as pl
from jax.experimental.pallas import tpu as pltpu
from jax.experimental.pallas import tpu_sc as plsc

BATCH = 16384
EMBED_DIM = 32
NUM_CORES = 2
NUM_SUBCORES = 16
NUM_WORKERS = NUM_CORES * NUM_SUBCORES
BPW = BATCH // NUM_WORKERS
CHUNK = 128
NCHUNKS = BPW // CHUNK
LANES = 16

_mesh = plsc.VectorSubcoreMesh(core_axis_name="c", subcore_axis_name="s")

_GATHER_DNUMS = lax.GatherDimensionNumbers(
    offset_dims=(), collapsed_slice_dims=(0,), start_index_map=(0,))


def _lane_sum(x, lane):
    for k in (1, 2, 4, 8):
        idx = (lane ^ k).reshape(LANES, 1)
        x = x + lax.gather(x, idx, _GATHER_DNUMS, (1,),
                           mode=lax.GatherScatterMode.PROMISE_IN_BOUNDS)
    return x


@functools.partial(
    pl.kernel,
    mesh=_mesh,
    compiler_params=pltpu.CompilerParams(use_tc_tiling_on_sc=False),
    out_type=jax.ShapeDtypeStruct((BATCH,), jnp.float32),
    scratch_types=[
        pltpu.VMEM((NCHUNKS, CHUNK), jnp.int32),
        pltpu.VMEM((NCHUNKS, CHUNK), jnp.int32),
        pltpu.VMEM((BPW, EMBED_DIM), jnp.float32),
        pltpu.VMEM((BPW, EMBED_DIM), jnp.float32),
        pltpu.VMEM((BPW,), jnp.float32),
        pltpu.SemaphoreType.DMA,
    ],
)
def _sc_dot(uid_hbm, iid_hbm, utab_hbm, itab_hbm, out_hbm,
            uidx_v, iidx_v, urows_v, irows_v, out_v, sem):
    wid = lax.axis_index("s") * NUM_CORES + lax.axis_index("c")
    base = wid * BPW

    for c in range(NCHUNKS):
        off = pl.ds(base + c * CHUNK, CHUNK)
        pltpu.sync_copy(uid_hbm.at[off], uidx_v.at[c])
        pltpu.sync_copy(iid_hbm.at[off], iidx_v.at[c])

    copies = []
    for c in range(NCHUNKS):
        dst = pl.ds(c * CHUNK, CHUNK)
        copies.append(
            pltpu.async_copy(utab_hbm.at[uidx_v.at[c]], urows_v.at[dst], sem))
        copies.append(
            pltpu.async_copy(itab_hbm.at[iidx_v.at[c]], irows_v.at[dst], sem))
    for cp in copies:
        cp.wait()

    lane = lax.iota(jnp.int32, LANES)

    def group_body(g, _):
        base_r = g * LANES
        acc = jnp.zeros((LANES,), jnp.float32)
        for j in range(LANES):
            r = base_r + j
            u0 = urows_v[r, pl.ds(0, LANES)]
            u1 = urows_v[r, pl.ds(LANES, LANES)]
            v0 = irows_v[r, pl.ds(0, LANES)]
            v1 = irows_v[r, pl.ds(LANES, LANES)]
            s = _lane_sum(u0 * v0 + u1 * v1, lane)
            acc = jnp.where(lane == j, s, acc)
        out_v[pl.ds(base_r, LANES)] = acc
        return 0

    lax.fori_loop(0, BPW // LANES, group_body, 0)

    pltpu.sync_copy(out_v, out_hbm.at[pl.ds(base, BPW)])


def kernel(user_ids, item_ids, user_table, item_table):
    return _sc_dot(user_ids, item_ids, user_table, item_table)
